# baseline (speedup 1.0000x reference)
import functools

import jax
import jax.numpy as jnp
from jax import lax
from jax.experimental import pallas as pl
from jax.experimental.pallas import tpu as pltpu
from jax.experimental.pallas import tpu_sc as plsc

EMB_DIM = 128
NUM_CORES = 2
NUM_SUBCORES = 16
NUM_WORKERS = NUM_CORES * NUM_SUBCORES
T_BUF = 8


@functools.partial(jax.jit, static_argnames=("n_tok", "seq"))
def _embedding_gather(token_ids_3d, table, *, n_tok, seq):
    tok_per_w = n_tok // NUM_WORKERS
    n_grp = tok_per_w // T_BUF
    mesh = plsc.VectorSubcoreMesh(core_axis_name="c", subcore_axis_name="s")

    @functools.partial(
        pl.kernel,
        mesh=mesh,
        out_type=jax.ShapeDtypeStruct((n_tok, seq, EMB_DIM), jnp.float32),
        scratch_types=[
            pltpu.VMEM((tok_per_w, seq), jnp.int32),
            pltpu.VMEM((T_BUF, seq, EMB_DIM), jnp.float32),
            pltpu.SemaphoreType.DMA,
        ],
    )
    def k(idx_hbm, table_hbm, out_hbm, idx_v, buf, gsem):
        wid = lax.axis_index("s") * NUM_CORES + lax.axis_index("c")
        tok0 = wid * tok_per_w
        pltpu.sync_copy(idx_hbm.at[wid], idx_v)

        def body(g, _):
            t0 = g * T_BUF
            for t in range(T_BUF):
                pltpu.async_copy(
                    table_hbm.at[idx_v.at[t0 + t]], buf.at[t], gsem
                )
            for t in range(T_BUF):
                pltpu.make_async_copy(
                    table_hbm.at[idx_v.at[t0 + t]], buf.at[t], gsem
                ).wait()
            pltpu.sync_copy(buf, out_hbm.at[pl.ds(tok0 + t0, T_BUF)])
            return 0

        lax.fori_loop(0, n_grp, body, 0)

    return k(token_ids_3d, table)


N_SPLIT = 4


def kernel(token_ids, embedding_map):
    n_tok, seq = token_ids.shape
    assert n_tok % (N_SPLIT * NUM_WORKERS * T_BUF) == 0
    tok_per_call = n_tok // N_SPLIT
    idx4 = token_ids.astype(jnp.int32).reshape(
        N_SPLIT, NUM_WORKERS, tok_per_call // NUM_WORKERS, seq
    )
    parts = [
        _embedding_gather(idx4[i], embedding_map, n_tok=tok_per_call, seq=seq)
        for i in range(N_SPLIT)
    ]
    return jnp.concatenate(parts, axis=0)


# speedup vs baseline: 1.7314x; 1.7314x over previous
import functools

import jax
import jax.numpy as jnp
from jax import lax
from jax.experimental import pallas as pl
from jax.experimental.pallas import tpu as pltpu
from jax.experimental.pallas import tpu_sc as plsc

EMB_DIM = 128
NUM_CORES = 2
NUM_SUBCORES = 16
NUM_WORKERS = NUM_CORES * NUM_SUBCORES
T_BUF = 8


@functools.partial(jax.jit, static_argnames=("n_tok", "seq"))
def _embedding_gather(token_ids_3d, table, *, n_tok, seq):
    tok_per_w = n_tok // NUM_WORKERS
    n_grp = tok_per_w // T_BUF
    mesh = plsc.VectorSubcoreMesh(core_axis_name="c", subcore_axis_name="s")

    @functools.partial(
        pl.kernel,
        mesh=mesh,
        out_type=jax.ShapeDtypeStruct((n_tok, seq, EMB_DIM), jnp.float32),
        compiler_params=pltpu.CompilerParams(use_tc_tiling_on_sc=True),
        scratch_types=[
            pltpu.VMEM((tok_per_w, seq), jnp.int32),
            pltpu.VMEM((T_BUF, seq, EMB_DIM), jnp.float32),
            pltpu.SemaphoreType.DMA,
        ],
    )
    def k(idx_hbm, table_hbm, out_hbm, idx_v, buf, gsem):
        wid = lax.axis_index("s") * NUM_CORES + lax.axis_index("c")
        tok0 = wid * tok_per_w
        pltpu.sync_copy(idx_hbm.at[wid], idx_v)

        def body(g, _):
            t0 = g * T_BUF
            for t in range(T_BUF):
                pltpu.async_copy(
                    table_hbm.at[idx_v.at[t0 + t]], buf.at[t], gsem
                )
            for t in range(T_BUF):
                pltpu.make_async_copy(
                    table_hbm.at[idx_v.at[t0 + t]], buf.at[t], gsem
                ).wait()
            pltpu.sync_copy(buf, out_hbm.at[pl.ds(tok0 + t0, T_BUF)])
            return 0

        lax.fori_loop(0, n_grp, body, 0)

    return k(token_ids_3d, table)


def kernel(token_ids, embedding_map):
    n_tok, seq = token_ids.shape
    assert n_tok % (NUM_WORKERS * T_BUF) == 0
    idx3 = token_ids.astype(jnp.int32).reshape(NUM_WORKERS, n_tok // NUM_WORKERS, seq)
    return _embedding_gather(idx3, embedding_map, n_tok=n_tok, seq=seq)


# speedup vs baseline: 2.6370x; 1.5230x over previous
import functools

import jax
import jax.numpy as jnp
from jax import lax
from jax.experimental import pallas as pl
from jax.experimental.pallas import tpu as pltpu
from jax.experimental.pallas import tpu_sc as plsc

EMB_DIM = 128
NUM_CORES = 2
NUM_SUBCORES = 16
NUM_WORKERS = NUM_CORES * NUM_SUBCORES
TOK_CHUNK = 128


@functools.partial(jax.jit, static_argnames=("n_tok", "seq"))
def _embedding_gather(idx_wsj, table, *, n_tok, seq):
    mesh = plsc.VectorSubcoreMesh(core_axis_name="c", subcore_axis_name="s")

    @functools.partial(
        pl.kernel,
        mesh=mesh,
        out_type=jax.ShapeDtypeStruct((seq, n_tok, EMB_DIM), jnp.float32),
        scratch_types=[
            pltpu.VMEM((seq, TOK_CHUNK), jnp.int32),
            pltpu.VMEM((TOK_CHUNK, EMB_DIM), jnp.float32),
            pltpu.VMEM((TOK_CHUNK, EMB_DIM), jnp.float32),
            pltpu.SemaphoreType.DMA,
            pltpu.SemaphoreType.DMA,
            pltpu.SemaphoreType.DMA,
            pltpu.SemaphoreType.DMA,
        ],
    )
    def k(idx_hbm, table_hbm, out_hbm, idx_v, buf0, buf1, g0, g1, s0, s1):
        wid = lax.axis_index("s") * NUM_CORES + lax.axis_index("c")
        tok0 = wid * TOK_CHUNK
        pltpu.sync_copy(idx_hbm.at[wid], idx_v)

        bufs = (buf0, buf1)
        gsems = (g0, g1)
        ssems = (s0, s1)

        def out_at(c):
            return out_hbm.at[c, pl.ds(tok0, TOK_CHUNK)]

        def gather(c, b):
            pltpu.async_copy(table_hbm.at[idx_v.at[c]], bufs[b], gsems[b])

        def wait_gather(c, b):
            pltpu.make_async_copy(
                table_hbm.at[idx_v.at[c]], bufs[b], gsems[b]
            ).wait()

        def store(c, b):
            pltpu.async_copy(bufs[b], out_at(c), ssems[b])

        def wait_store(c, b):
            pltpu.make_async_copy(bufs[b], out_at(c), ssems[b]).wait()

        n_pairs = seq // 2
        gather(0, 0)

        def body(p, _):
            for b in range(2):
                c = 2 * p + b
                other = 1 - b
                wait_gather(c, b)
                if b == 1:
                    wait_store(c, other)
                else:
                    @pl.when(p > 0)
                    def _():
                        wait_store(c, other)
                if b == 0:
                    gather(c + 1, other)
                else:
                    @pl.when(p < n_pairs - 1)
                    def _():
                        gather(c + 1, other)
                store(c, b)
            return 0

        lax.fori_loop(0, n_pairs, body, 0)
        wait_store(seq - 1, 1)

    return k(idx_wsj, table)


def kernel(token_ids, embedding_map):
    n_tok, seq = token_ids.shape
    assert n_tok % (NUM_WORKERS * TOK_CHUNK) == 0 and seq % 2 == 0
    idx_wsj = jnp.transpose(
        token_ids.astype(jnp.int32).reshape(NUM_WORKERS, TOK_CHUNK, seq),
        (0, 2, 1),
    )
    out = _embedding_gather(idx_wsj, embedding_map, n_tok=n_tok, seq=seq)
    return jnp.transpose(out, (1, 0, 2))


# speedup vs baseline: 3.2877x; 1.2467x over previous
import functools

import jax
import jax.numpy as jnp
from jax import lax
from jax.experimental import pallas as pl
from jax.experimental.pallas import tpu as pltpu
from jax.experimental.pallas import tpu_sc as plsc

EMB_DIM = 128
NUM_CORES = 2
NUM_SUBCORES = 16
NUM_WORKERS = NUM_CORES * NUM_SUBCORES
TOK_CHUNK = 128


@functools.partial(jax.jit, static_argnames=("n_tok", "seq"))
def _embedding_gather(idx_wsj, table, *, n_tok, seq):
    mesh = plsc.VectorSubcoreMesh(core_axis_name="c", subcore_axis_name="s")

    @functools.partial(
        pl.kernel,
        mesh=mesh,
        out_type=jax.ShapeDtypeStruct((seq, n_tok, EMB_DIM), jnp.float32),
        scratch_types=[
            pltpu.VMEM((seq, TOK_CHUNK), jnp.int32),
            pltpu.VMEM((5, TOK_CHUNK, EMB_DIM), jnp.float32),
            pltpu.SemaphoreType.DMA,
            pltpu.SemaphoreType.DMA,
            pltpu.SemaphoreType.DMA,
            pltpu.SemaphoreType.DMA,
            pltpu.SemaphoreType.DMA,
            pltpu.SemaphoreType.DMA,
            pltpu.SemaphoreType.DMA,
            pltpu.SemaphoreType.DMA,
            pltpu.SemaphoreType.DMA,
            pltpu.SemaphoreType.DMA,
        ],
    )
    def k(idx_hbm, table_hbm, out_hbm, idx_v, bufs, *sems):
        wid = lax.axis_index("s") * NUM_CORES + lax.axis_index("c")
        tok0 = wid * TOK_CHUNK
        pltpu.sync_copy(idx_hbm.at[wid], idx_v)

        gsems = sems[:5]
        ssems = sems[5:]

        def out_at(c):
            return out_hbm.at[c, pl.ds(tok0, TOK_CHUNK)]

        def gather(c, b):
            pltpu.async_copy(table_hbm.at[idx_v.at[c]], bufs.at[b], gsems[b])

        def wait_gather(c, b):
            pltpu.make_async_copy(
                table_hbm.at[idx_v.at[c]], bufs.at[b], gsems[b]
            ).wait()

        def store(c, b):
            pltpu.async_copy(bufs.at[b], out_at(c), ssems[b])

        def wait_store(c, b):
            pltpu.make_async_copy(bufs.at[b], out_at(c), ssems[b]).wait()

        n_grp = seq // 5
        for c in range(4):
            gather(c, c)

        def body(i, _):
            for b in range(5):
                c = 5 * i + b
                wait_gather(c, b)
                store(c, b)
                nb = (b + 4) % 5
                if b == 0:
                    @pl.when(i > 0)
                    def _():
                        wait_store(5 * i - 1, nb)
                    gather(c + 4, nb)
                else:
                    @pl.when(i < n_grp - 1)
                    def _():
                        wait_store(c - 1, nb)
                        gather(c + 4, nb)
            return 0

        lax.fori_loop(0, n_grp, body, 0)
        for c in range(seq - 5, seq):
            wait_store(c, c % 5)

    return k(idx_wsj, table)


def kernel(token_ids, embedding_map):
    n_tok, seq = token_ids.shape
    assert n_tok % (NUM_WORKERS * TOK_CHUNK) == 0 and seq % 2 == 0
    idx_wsj = jnp.transpose(
        token_ids.astype(jnp.int32).reshape(NUM_WORKERS, TOK_CHUNK, seq),
        (0, 2, 1),
    )
    out = _embedding_gather(idx_wsj, embedding_map, n_tok=n_tok, seq=seq)
    return jnp.transpose(out, (1, 0, 2))


# speedup vs baseline: 3.3018x; 1.0043x over previous
import functools

import jax
import jax.numpy as jnp
from jax import lax
from jax.experimental import pallas as pl
from jax.experimental.pallas import tpu as pltpu
from jax.experimental.pallas import tpu_sc as plsc

EMB_DIM = 128
NUM_CORES = 2
NUM_SUBCORES = 16
NUM_WORKERS = NUM_CORES * NUM_SUBCORES
TOK_CHUNK = 128
NBUF = 6


@functools.partial(jax.jit, static_argnames=("n_tok", "seq"))
def _embedding_gather(idx_wsj, table, *, n_tok, seq):
    mesh = plsc.VectorSubcoreMesh(core_axis_name="c", subcore_axis_name="s")

    @functools.partial(
        pl.kernel,
        mesh=mesh,
        out_type=jax.ShapeDtypeStruct((seq, n_tok, EMB_DIM), jnp.float32),
        scratch_types=[
            pltpu.VMEM((seq, TOK_CHUNK), jnp.int32),
            pltpu.VMEM((NBUF, TOK_CHUNK, EMB_DIM), jnp.float32),
        ]
        + [pltpu.SemaphoreType.DMA] * (2 * NBUF),
    )
    def k(idx_hbm, table_hbm, out_hbm, idx_v, bufs, *sems):
        wid = lax.axis_index("s") * NUM_CORES + lax.axis_index("c")
        tok0 = wid * TOK_CHUNK
        pltpu.sync_copy(idx_hbm.at[wid], idx_v)

        gsems = sems[:NBUF]
        ssems = sems[NBUF:]

        def out_at(c):
            return out_hbm.at[c, pl.ds(tok0, TOK_CHUNK)]

        def gather(c, b):
            pltpu.async_copy(table_hbm.at[idx_v.at[c]], bufs.at[b], gsems[b])

        def wait_gather(c, b):
            pltpu.make_async_copy(
                table_hbm.at[idx_v.at[c]], bufs.at[b], gsems[b]
            ).wait()

        def store(c, b):
            pltpu.async_copy(bufs.at[b], out_at(c), ssems[b])

        def wait_store(c, b):
            pltpu.make_async_copy(bufs.at[b], out_at(c), ssems[b]).wait()

        look = NBUF - 1
        n_grp = seq // NBUF
        n_main = n_grp * NBUF
        tail = seq - n_main
        assert tail < NBUF
        b_cut = (seq - 1 - look) - NBUF * (n_grp - 1)
        assert 0 <= b_cut < NBUF

        for c in range(look):
            gather(c, c)

        def body(i, _):
            for b in range(NBUF):
                c = NBUF * i + b
                wait_gather(c, b)
                store(c, b)
                nb = (b + look) % NBUF
                if b == 0:
                    @pl.when(i > 0)
                    def _():
                        wait_store(NBUF * i - 1, nb)
                    gather(c + look, nb)
                elif b <= b_cut:
                    wait_store(c - 1, nb)
                    gather(c + look, nb)
                else:
                    @pl.when(i < n_grp - 1)
                    def _():
                        wait_store(c - 1, nb)
                        gather(c + look, nb)
            return 0

        lax.fori_loop(0, n_grp, body, 0)
        for c in range(n_main, seq):
            wait_gather(c, c % NBUF)
            store(c, c % NBUF)
        for c in range(seq - NBUF, seq):
            wait_store(c, c % NBUF)

    return k(idx_wsj, table)


def kernel(token_ids, embedding_map):
    n_tok, seq = token_ids.shape
    assert n_tok % (NUM_WORKERS * TOK_CHUNK) == 0 and seq % 2 == 0
    idx_wsj = jnp.transpose(
        token_ids.astype(jnp.int32).reshape(NUM_WORKERS, TOK_CHUNK, seq),
        (0, 2, 1),
    )
    out = _embedding_gather(idx_wsj, embedding_map, n_tok=n_tok, seq=seq)
    return jnp.transpose(out, (1, 0, 2))


# speedup vs baseline: 3.3191x; 1.0052x over previous
import functools

import jax
import jax.numpy as jnp
from jax import lax
from jax.experimental import pallas as pl
from jax.experimental.pallas import tpu as pltpu
from jax.experimental.pallas import tpu_sc as plsc

EMB_DIM = 128
NUM_CORES = 2
NUM_SUBCORES = 16
NUM_WORKERS = NUM_CORES * NUM_SUBCORES
BLK = 128
TOK_CHUNK = 64
PER_PLANE = BLK // TOK_CHUNK
NBUF = 10


@functools.partial(jax.jit, static_argnames=("n_tok", "seq"))
def _embedding_gather(idx_wsj, table, *, n_tok, seq):
    mesh = plsc.VectorSubcoreMesh(core_axis_name="c", subcore_axis_name="s")

    @functools.partial(
        pl.kernel,
        mesh=mesh,
        out_type=jax.ShapeDtypeStruct((seq, n_tok, EMB_DIM), jnp.float32),
        scratch_types=[
            pltpu.VMEM((seq, BLK), jnp.int32),
            pltpu.VMEM((NBUF, TOK_CHUNK, EMB_DIM), jnp.float32),
        ]
        + [pltpu.SemaphoreType.DMA] * (2 * NBUF),
    )
    def k(idx_hbm, table_hbm, out_hbm, idx_v, bufs, *sems):
        wid = lax.axis_index("s") * NUM_CORES + lax.axis_index("c")
        tok0 = wid * BLK
        pltpu.sync_copy(idx_hbm.at[wid], idx_v)

        gsems = sems[:NBUF]
        ssems = sems[NBUF:]

        def out_at(c):
            plane = c // PER_PLANE
            off = (c % PER_PLANE) * TOK_CHUNK
            return out_hbm.at[plane, pl.ds(tok0 + off, TOK_CHUNK)]

        def idx_at(c):
            plane = c // PER_PLANE
            off = (c % PER_PLANE) * TOK_CHUNK
            return idx_v.at[plane, pl.ds(off, TOK_CHUNK)]

        def gather(c, b):
            pltpu.async_copy(table_hbm.at[idx_at(c)], bufs.at[b], gsems[b])

        def wait_gather(c, b):
            pltpu.make_async_copy(
                table_hbm.at[idx_at(c)], bufs.at[b], gsems[b]
            ).wait()

        def store(c, b):
            pltpu.async_copy(bufs.at[b], out_at(c), ssems[b])

        def wait_store(c, b):
            pltpu.make_async_copy(bufs.at[b], out_at(c), ssems[b]).wait()

        n_chunks = seq * PER_PLANE
        look = NBUF - 1
        n_grp = n_chunks // NBUF
        n_main = n_grp * NBUF
        tail = n_chunks - n_main
        assert tail < NBUF
        b_cut = (n_chunks - 1 - look) - NBUF * (n_grp - 1)
        assert 0 <= b_cut < NBUF

        for c in range(look):
            gather(c, c)

        def body(i, _):
            for b in range(NBUF):
                c = NBUF * i + b
                wait_gather(c, b)
                store(c, b)
                nb = (b + look) % NBUF
                if b == 0:
                    @pl.when(i > 0)
                    def _():
                        wait_store(NBUF * i - 1, nb)
                    gather(c + look, nb)
                elif b <= b_cut:
                    wait_store(c - 1, nb)
                    gather(c + look, nb)
                else:
                    @pl.when(i < n_grp - 1)
                    def _():
                        wait_store(c - 1, nb)
                        gather(c + look, nb)
            return 0

        lax.fori_loop(0, n_grp, body, 0)
        for c in range(n_main, n_chunks):
            wait_gather(c, c % NBUF)
            store(c, c % NBUF)
        for c in range(n_chunks - NBUF, n_chunks):
            wait_store(c, c % NBUF)

    return k(idx_wsj, table)


def kernel(token_ids, embedding_map):
    n_tok, seq = token_ids.shape
    assert n_tok % (NUM_WORKERS * BLK) == 0 and seq % 2 == 0
    idx_wsj = jnp.transpose(
        token_ids.astype(jnp.int32).reshape(NUM_WORKERS, BLK, seq),
        (0, 2, 1),
    )
    out = _embedding_gather(idx_wsj, embedding_map, n_tok=n_tok, seq=seq)
    return jnp.transpose(out, (1, 0, 2))


# speedup vs baseline: 3.3317x; 1.0038x over previous
import functools

import jax
import jax.numpy as jnp
from jax import lax
from jax.experimental import pallas as pl
from jax.experimental.pallas import tpu as pltpu
from jax.experimental.pallas import tpu_sc as plsc

EMB_DIM = 128
NUM_CORES = 2
NUM_SUBCORES = 16
NUM_WORKERS = NUM_CORES * NUM_SUBCORES
BLK = 128
TOK_CHUNK = 64
PER_PLANE = BLK // TOK_CHUNK
NBUF = 10


@functools.partial(jax.jit, static_argnames=("n_tok", "seq"))
def _embedding_gather(idx_wsj, table, *, n_tok, seq):
    mesh = plsc.VectorSubcoreMesh(core_axis_name="c", subcore_axis_name="s")

    @functools.partial(
        pl.kernel,
        mesh=mesh,
        out_type=jax.ShapeDtypeStruct((seq, n_tok, EMB_DIM), jnp.float32),
        scratch_types=[
            pltpu.VMEM((seq, BLK), jnp.int32),
            pltpu.VMEM((NBUF, TOK_CHUNK, EMB_DIM), jnp.float32),
            pltpu.SemaphoreType.DMA((NBUF,)),
            pltpu.SemaphoreType.DMA((NBUF,)),
        ],
    )
    def k(idx_hbm, table_hbm, out_hbm, idx_v, bufs, gsems, ssems):
        wid = lax.axis_index("s") * NUM_CORES + lax.axis_index("c")
        tok0 = wid * BLK
        pltpu.sync_copy(idx_hbm.at[wid], idx_v)

        def out_at(c):
            plane = c // PER_PLANE
            off = (c % PER_PLANE) * TOK_CHUNK
            return out_hbm.at[plane, pl.ds(tok0 + off, TOK_CHUNK)]

        def idx_at(c):
            plane = c // PER_PLANE
            off = (c % PER_PLANE) * TOK_CHUNK
            return idx_v.at[plane, pl.ds(off, TOK_CHUNK)]

        def gather(c, b):
            pltpu.async_copy(table_hbm.at[idx_at(c)], bufs.at[b], gsems.at[b])

        def wait_gather(c, b):
            pltpu.make_async_copy(
                table_hbm.at[idx_at(c)], bufs.at[b], gsems.at[b]
            ).wait()

        def store(c, b):
            pltpu.async_copy(bufs.at[b], out_at(c), ssems.at[b])

        def wait_store(c, b):
            pltpu.make_async_copy(bufs.at[b], out_at(c), ssems.at[b]).wait()

        n_chunks = seq * PER_PLANE
        look = NBUF - 1

        for c in range(look):
            gather(c, c)

        def body(c, _):
            b = lax.rem(c, NBUF)
            wait_gather(c, b)
            store(c, b)
            n = c + look
            nb = lax.rem(n, NBUF)

            @pl.when(n < n_chunks)
            def _():
                @pl.when(c >= 1)
                def _():
                    wait_store(c - 1, nb)
                gather(n, nb)

            return 0

        lax.fori_loop(0, n_chunks, body, 0)
        for c in range(n_chunks - NBUF, n_chunks):
            wait_store(c, c % NBUF)

    return k(idx_wsj, table)


def kernel(token_ids, embedding_map):
    n_tok, seq = token_ids.shape
    assert n_tok % (NUM_WORKERS * BLK) == 0 and seq % 2 == 0
    idx_wsj = jnp.transpose(
        token_ids.astype(jnp.int32).reshape(NUM_WORKERS, BLK, seq),
        (0, 2, 1),
    )
    out = _embedding_gather(idx_wsj, embedding_map, n_tok=n_tok, seq=seq)
    return jnp.transpose(out, (1, 0, 2))


# speedup vs baseline: 3.3398x; 1.0024x over previous
import functools

import jax
import jax.numpy as jnp
from jax import lax
from jax.experimental import pallas as pl
from jax.experimental.pallas import tpu as pltpu
from jax.experimental.pallas import tpu_sc as plsc

EMB_DIM = 128
NUM_CORES = 2
NUM_SUBCORES = 16
NUM_WORKERS = NUM_CORES * NUM_SUBCORES
BLK = 128
TOK_CHUNK = 64
PER_PLANE = BLK // TOK_CHUNK
NBUF = 10


@functools.partial(jax.jit, static_argnames=("n_tok", "seq"))
def _embedding_gather(idx_wsj, table, *, n_tok, seq):
    mesh = plsc.VectorSubcoreMesh(core_axis_name="c", subcore_axis_name="s")

    @functools.partial(
        pl.kernel,
        mesh=mesh,
        out_type=jax.ShapeDtypeStruct((seq, n_tok, EMB_DIM), jnp.float32),
        scratch_types=[
            pltpu.VMEM((seq, BLK), jnp.int32),
            pltpu.VMEM((NBUF, TOK_CHUNK, EMB_DIM), jnp.float32),
            pltpu.SemaphoreType.DMA((NBUF,)),
            pltpu.SemaphoreType.DMA((NBUF,)),
        ],
    )
    def k(idx_hbm, table_hbm, out_hbm, idx_v, bufs, gsems, ssems):
        wid = lax.axis_index("s") * NUM_CORES + lax.axis_index("c")
        tok0 = wid * BLK
        pltpu.sync_copy(idx_hbm.at[:, wid], idx_v)

        def out_at(c):
            plane = c // PER_PLANE
            off = (c % PER_PLANE) * TOK_CHUNK
            return out_hbm.at[plane, pl.ds(tok0 + off, TOK_CHUNK)]

        def idx_at(c):
            plane = c // PER_PLANE
            off = (c % PER_PLANE) * TOK_CHUNK
            return idx_v.at[plane, pl.ds(off, TOK_CHUNK)]

        def gather(c, b):
            pltpu.async_copy(table_hbm.at[idx_at(c)], bufs.at[b], gsems.at[b])

        def wait_gather(c, b):
            pltpu.make_async_copy(
                table_hbm.at[idx_at(c)], bufs.at[b], gsems.at[b]
            ).wait()

        def store(c, b):
            pltpu.async_copy(bufs.at[b], out_at(c), ssems.at[b])

        def wait_store(c, b):
            pltpu.make_async_copy(bufs.at[b], out_at(c), ssems.at[b]).wait()

        n_chunks = seq * PER_PLANE
        look = NBUF - 1

        for c in range(look):
            gather(c, c)

        def body(c, _):
            b = lax.rem(c, NBUF)
            wait_gather(c, b)
            store(c, b)
            n = c + look
            nb = lax.rem(n, NBUF)

            @pl.when(n < n_chunks)
            def _():
                @pl.when(c >= 1)
                def _():
                    wait_store(c - 1, nb)
                gather(n, nb)

            return 0

        lax.fori_loop(0, n_chunks, body, 0)
        for c in range(n_chunks - NBUF, n_chunks):
            wait_store(c, c % NBUF)

    return k(idx_wsj, table)


def kernel(token_ids, embedding_map):
    n_tok, seq = token_ids.shape
    assert n_tok % (NUM_WORKERS * BLK) == 0 and seq % 2 == 0
    idx_wsj = jnp.transpose(token_ids.astype(jnp.int32), (1, 0)).reshape(
        seq, NUM_WORKERS, BLK
    )
    out = _embedding_gather(idx_wsj, embedding_map, n_tok=n_tok, seq=seq)
    return jnp.transpose(out, (1, 0, 2))
